# baseline (speedup 1.0000x reference)
import jax
import jax.numpy as jnp
from jax import lax
from jax.experimental import pallas as pl
from jax.experimental.pallas import tpu as pltpu

N = 8192
D = 128


def _patch_kernel(pos_ref, e_ref, emb_ref, w_in, emb_out, w_out,
                  sr_ref, stripe_ref, row_ref, sem_in, sem_out, sem_row,
                  sem_emb):
    del w_in
    pos = pos_ref[0]
    stripe0 = (pos // 128) * 128

    cp_in = pltpu.make_async_copy(
        w_out.at[:, pl.ds(stripe0, 128)], stripe_ref, sem_in)
    cp_in.start()

    E = emb_ref[...]
    ev = e_ref[...]
    dots_c = lax.dot_general(E, ev, (((1,), (1,)), ((), ())),
                             preferred_element_type=jnp.float32)
    n2_c = jnp.sum(E * E, axis=1, keepdims=True)
    sc = dots_c / (jnp.sqrt(n2_c) + 1e-8)
    dots_r = lax.dot_general(ev, E, (((1,), (1,)), ((), ())),
                             preferred_element_type=jnp.float32)
    ones = jnp.ones((1, D), jnp.float32)
    n2_r = lax.dot_general(ones, E * E, (((1,), (1,)), ((), ())),
                           preferred_element_type=jnp.float32)
    sr_ref[...] = dots_r / (jnp.sqrt(n2_r) + 1e-8)

    cp_row = pltpu.make_async_copy(sr_ref, w_out.at[pl.ds(pos, 1), :],
                                   sem_row)
    cp_row.start()

    row_ref[...] = ev
    cp_emb = pltpu.make_async_copy(row_ref, emb_out.at[pl.ds(pos, 1), :],
                                   sem_emb)
    cp_emb.start()

    cp_in.wait()
    S = stripe_ref[...]
    cols = lax.broadcasted_iota(jnp.int32, (N, 128), 1) + stripe0
    rows = lax.broadcasted_iota(jnp.int32, (N, 128), 0)
    S = jnp.where(cols == pos, sc, S)
    S = jnp.where(rows == pos, sr_ref[0, pl.ds(stripe0, 128)][None, :], S)
    stripe_ref[...] = S
    cp_out = pltpu.make_async_copy(
        stripe_ref, w_out.at[:, pl.ds(stripe0, 128)], sem_out)
    cp_out.start()

    cp_out.wait()
    cp_row.wait()
    cp_emb.wait()


def kernel(experience_embeddings, associative_weights, experience,
           temporal_context, position):
    del temporal_context
    pos = jnp.asarray(position, jnp.int32).reshape(1)
    e2 = experience.reshape(1, D)

    new_emb, new_w = pl.pallas_call(
        _patch_kernel,
        grid=(1,),
        out_shape=(jax.ShapeDtypeStruct((N, D), jnp.float32),
                   jax.ShapeDtypeStruct((N, N), jnp.float32)),
        in_specs=[pl.BlockSpec(memory_space=pltpu.SMEM),
                  pl.BlockSpec((1, D), lambda i: (0, 0)),
                  pl.BlockSpec((N, D), lambda i: (0, 0)),
                  pl.BlockSpec(memory_space=pl.ANY)],
        out_specs=(pl.BlockSpec(memory_space=pl.ANY),
                   pl.BlockSpec(memory_space=pl.ANY)),
        scratch_shapes=[pltpu.VMEM((1, N), jnp.float32),
                        pltpu.VMEM((N, 128), jnp.float32),
                        pltpu.VMEM((1, D), jnp.float32),
                        pltpu.SemaphoreType.DMA,
                        pltpu.SemaphoreType.DMA,
                        pltpu.SemaphoreType.DMA,
                        pltpu.SemaphoreType.DMA],
        input_output_aliases={2: 0, 3: 1},
    )(pos, e2, experience_embeddings, associative_weights)

    return (new_emb, new_w)


# speedup vs baseline: 1.0365x; 1.0365x over previous
import jax
import jax.numpy as jnp
from jax import lax
from jax.experimental import pallas as pl
from jax.experimental.pallas import tpu as pltpu

N = 8192
D = 128
BLK = 256


def _fused_kernel(pos_ref, e_ref, embf_ref, emb_ref, w_ref,
                  new_emb_ref, out_ref):
    i = pl.program_id(0)
    pos = pos_ref[0]
    ev = e_ref[...]

    E_blk = emb_ref[...]
    dots_c = lax.dot_general(E_blk, ev, (((1,), (1,)), ((), ())),
                             preferred_element_type=jnp.float32)
    n2_c = jnp.sum(E_blk * E_blk, axis=1, keepdims=True)
    sc_blk = dots_c / (jnp.sqrt(n2_c) + 1e-8)

    rows_d = lax.broadcasted_iota(jnp.int32, (BLK, D), 0) + i * BLK
    new_emb_ref[...] = jnp.where(rows_d == pos, ev, E_blk)

    W = w_ref[...]
    rows = lax.broadcasted_iota(jnp.int32, (BLK, N), 0) + i * BLK
    cols = lax.broadcasted_iota(jnp.int32, (BLK, N), 1)
    W = jnp.where(cols == pos, sc_blk, W)
    out_ref[...] = W

    @pl.when((pos >= i * BLK) & (pos < (i + 1) * BLK))
    def _():
        E = embf_ref[...]
        dots_r = lax.dot_general(ev, E, (((1,), (1,)), ((), ())),
                                 preferred_element_type=jnp.float32)
        ones = jnp.ones((1, D), jnp.float32)
        n2_r = lax.dot_general(ones, E * E, (((1,), (1,)), ((), ())),
                               preferred_element_type=jnp.float32)
        sr = dots_r / (jnp.sqrt(n2_r) + 1e-8)
        out_ref[pl.ds(pos - i * BLK, 1), :] = sr


def kernel(experience_embeddings, associative_weights, experience,
           temporal_context, position):
    del temporal_context
    pos = jnp.asarray(position, jnp.int32).reshape(1)
    e2 = experience.reshape(1, D)

    new_emb, new_w = pl.pallas_call(
        _fused_kernel,
        grid=(N // BLK,),
        out_shape=(jax.ShapeDtypeStruct((N, D), jnp.float32),
                   jax.ShapeDtypeStruct((N, N), jnp.float32)),
        in_specs=[pl.BlockSpec(memory_space=pltpu.SMEM),
                  pl.BlockSpec((1, D), lambda i: (0, 0)),
                  pl.BlockSpec((N, D), lambda i: (0, 0)),
                  pl.BlockSpec((BLK, D), lambda i: (i, 0)),
                  pl.BlockSpec((BLK, N), lambda i: (i, 0))],
        out_specs=(pl.BlockSpec((BLK, D), lambda i: (i, 0)),
                   pl.BlockSpec((BLK, N), lambda i: (i, 0))),
        compiler_params=pltpu.CompilerParams(
            dimension_semantics=("parallel",)),
    )(pos, e2, experience_embeddings, experience_embeddings,
      associative_weights)

    return (new_emb, new_w)
